# baseline (speedup 1.0000x reference)
import jax
import jax.numpy as jnp
from jax import lax
from jax.experimental import pallas as pl
from jax.experimental.pallas import tpu as pltpu
from jax.experimental.pallas import tpu_sc as plsc

F32 = jnp.float32
NC, NS = 2, 16
NW = NC * NS
CH = 80


def _sc_mesh():
    return plsc.VectorSubcoreMesh(
        core_axis_name="c", subcore_axis_name="s",
        num_cores=NC, num_subcores=NS)


def _deg_call(n, rpt, dstp3d, dsts3d):
    blk = (n // NS) // 8 * 8
    tail = n - NS * blk
    nred = blk // 16
    inner = CH // 16

    def body(dstp, dsts, op0, op1, os0, os1,
             histp, hists, ip_all, is_all, red, redt, obuf, obuft,
             stgp, stgs, sem):
        c = lax.axis_index("c")
        s = lax.axis_index("s")
        wid = s * NC + c
        z16 = jnp.zeros((16,), F32)

        def zloop(i, carry):
            histp[pl.ds(i * 16, 16)] = z16
            hists[pl.ds(i * 16, 16)] = z16
            return carry
        lax.fori_loop(0, n // 16, zloop, 0)

        pltpu.sync_copy(dstp.at[wid], ip_all)
        pltpu.sync_copy(dsts.at[wid], is_all)

        for idx_all, hist in ((ip_all, histp), (is_all, hists)):
            def hloop(r, carry):
                for j in range(inner):
                    x = idx_all[r, pl.ds(j * 16, 16)]
                    cnt, last = plsc.scan_count(x)
                    plsc.addupdate_scatter(hist, [x], cnt.astype(F32),
                                           mask=last)
                return carry
            lax.fori_loop(0, rpt, hloop, 0)

        st_s = pl.multiple_of(s * n, 8)
        pltpu.sync_copy(histp, stgp.at[pl.ds(st_s, n)])
        pltpu.sync_copy(hists, stgs.at[pl.ds(st_s, n)])
        plsc.subcore_barrier()

        st = pl.multiple_of(s * blk, 8)
        for stg, o0, o1 in ((stgp, op0, op1), (stgs, os0, os1)):
            descs = [
                pltpu.async_copy(
                    stg.at[pl.ds(pl.multiple_of(k * n + st, 8), blk)],
                    red.at[k], sem)
                for k in range(NS)
            ]
            for de in descs:
                de.wait()

            def rloop(j, carry):
                v = red[0, pl.ds(j * 16, 16)]
                for k in range(1, NS):
                    v = v + red[k, pl.ds(j * 16, 16)]
                obuf[pl.ds(j * 16, 16)] = v
                return carry
            lax.fori_loop(0, nred, rloop, 0)

            @pl.when(c == 0)
            def _():
                pltpu.sync_copy(obuf, o0.at[pl.ds(st, blk)])

            @pl.when(c == 1)
            def _():
                pltpu.sync_copy(obuf, o1.at[pl.ds(st, blk)])

            @pl.when(s == NS - 1)
            def _():
                if tail:
                    tdescs = [
                        pltpu.async_copy(
                            stg.at[pl.ds(k * n + NS * blk, tail)],
                            redt.at[k], sem)
                        for k in range(NS)
                    ]
                    for de in tdescs:
                        de.wait()
                    v = redt[0, :]
                    for k in range(1, NS):
                        v = v + redt[k, :]
                    obuft[...] = v

                    @pl.when(c == 0)
                    def _():
                        pltpu.sync_copy(obuft, o0.at[pl.ds(NS * blk, tail)])

                    @pl.when(c == 1)
                    def _():
                        pltpu.sync_copy(obuft, o1.at[pl.ds(NS * blk, tail)])

    k = pl.kernel(
        body,
        out_type=[jax.ShapeDtypeStruct((n,), F32)] * 4,
        mesh=_sc_mesh(),
        compiler_params=pltpu.CompilerParams(use_tc_tiling_on_sc=False,
                                             needs_layout_passes=False),
        scratch_types=[
            pltpu.VMEM((n,), F32),
            pltpu.VMEM((n,), F32),
            pltpu.VMEM((rpt, CH), jnp.int32),
            pltpu.VMEM((rpt, CH), jnp.int32),
            pltpu.VMEM((NS, blk), F32),
            pltpu.VMEM((NS, 16), F32),
            pltpu.VMEM((blk,), F32),
            pltpu.VMEM((16,), F32),
            pltpu.VMEM_SHARED((NS * n,), F32),
            pltpu.VMEM_SHARED((NS * n,), F32),
            pltpu.SemaphoreType.DMA,
        ],
    )
    return k(dstp3d, dsts3d)


def _agg_call(n, d, g_nd, src3d, dst3d, zeros_nd):
    rpt = src3d.shape[1]
    K = 5
    nstg = rpt // K
    blk = (n // NS) // 8 * 8
    tail = n - NS * blk

    def body(g_hbm, src_h, dst_h, zn, out, acc,
             si_all, di_all, rows_a, rows_b, semg, sems):
        c = lax.axis_index("c")
        s = lax.axis_index("s")
        wid = s * NC + c
        rows = (rows_a, rows_b)

        st = pl.multiple_of(s * blk, 8)
        pltpu.sync_copy(zn.at[pl.ds(st, blk)], acc.at[pl.ds(st, blk)])

        @pl.when(s == NS - 1)
        def _():
            if tail:
                pltpu.sync_copy(zn.at[pl.ds(NS * blk, tail)],
                                acc.at[pl.ds(NS * blk, tail)])
        pltpu.sync_copy(src_h.at[wid], si_all)
        pltpu.sync_copy(dst_h.at[wid], di_all)
        plsc.subcore_barrier()

        def fire_gathers(t):
            dd = t % 2
            return [
                pltpu.async_copy(g_hbm.at[si_all.at[t * K + j]],
                                 rows[dd].at[j], semg)
                for j in range(K)
            ]

        def fire_scatters(t):
            dd = t % 2
            return [
                pltpu.async_copy(rows[dd].at[j],
                                 acc.at[di_all.at[t * K + j]], sems, add=True)
                for j in range(K)
            ]

        gat = {0: fire_gathers(0)}
        sca = {}
        for t in range(nstg):
            for de in gat.pop(t):
                de.wait()
            if t >= 1:
                for de in sca.pop(t - 1):
                    de.wait()
            if t + 1 < nstg:
                gat[t + 1] = fire_gathers(t + 1)
            sca[t] = fire_scatters(t)
        for de in sca.pop(nstg - 1):
            de.wait()

        plsc.subcore_barrier()
        pltpu.sync_copy(acc.at[pl.ds(st, blk)], out.at[c, pl.ds(st, blk)])

        @pl.when(s == NS - 1)
        def _():
            if tail:
                pltpu.sync_copy(acc.at[pl.ds(NS * blk, tail)],
                                out.at[c, pl.ds(NS * blk, tail)])

    k = pl.kernel(
        body,
        out_type=[jax.ShapeDtypeStruct((NC, n, d), F32)],
        mesh=_sc_mesh(),
        compiler_params=pltpu.CompilerParams(use_tc_tiling_on_sc=False),
        scratch_types=[
            pltpu.VMEM_SHARED((n, d), F32),
            pltpu.VMEM((rpt, CH), jnp.int32),
            pltpu.VMEM((rpt, CH), jnp.int32),
            pltpu.VMEM((K, CH, d), F32),
            pltpu.VMEM((K, CH, d), F32),
            pltpu.SemaphoreType.DMA,
            pltpu.SemaphoreType.DMA,
        ],
    )
    (out,) = k(g_nd, src3d, dst3d, zeros_nd)
    return out


def _pair_gather_call(n, qpad, d, zp, zs, e0_3d, e1_3d):
    rpt = e0_3d.shape[1]
    K = 4
    nstg = rpt // K

    def body(zp_h, zs_h, e0_h, e1_h, gp, gs,
             i0_all, i1_all, r0a, r0b, r1a, r1b, semg, semw):
        c = lax.axis_index("c")
        s = lax.axis_index("s")
        wid = s * NC + c
        r0 = (r0a, r0b)
        r1 = (r1a, r1b)

        pltpu.sync_copy(e0_h.at[wid], i0_all)
        pltpu.sync_copy(e1_h.at[wid], i1_all)

        def fire_gathers(t):
            dd = t % 2
            descs = []
            for j in range(K):
                descs.append(pltpu.async_copy(
                    zp_h.at[i0_all.at[t * K + j]],
                    r0[dd].at[pl.ds(j * CH, CH)], semg))
                descs.append(pltpu.async_copy(
                    zs_h.at[i1_all.at[t * K + j]],
                    r1[dd].at[pl.ds(j * CH, CH)], semg))
            return descs

        def fire_writes(t):
            dd = t % 2
            off = pl.multiple_of((wid * rpt + t * K) * CH, 8)
            return [
                pltpu.async_copy(r0[dd], gp.at[pl.ds(off, K * CH)], semw),
                pltpu.async_copy(r1[dd], gs.at[pl.ds(off, K * CH)], semw),
            ]

        gat = {0: fire_gathers(0)}
        wrt = {}
        for t in range(nstg):
            for de in gat.pop(t):
                de.wait()
            if t >= 1:
                for de in wrt.pop(t - 1):
                    de.wait()
            if t + 1 < nstg:
                gat[t + 1] = fire_gathers(t + 1)
            wrt[t] = fire_writes(t)
        for de in wrt.pop(nstg - 1):
            de.wait()

    k = pl.kernel(
        body,
        out_type=[jax.ShapeDtypeStruct((qpad, d), F32),
                  jax.ShapeDtypeStruct((qpad, d), F32)],
        mesh=_sc_mesh(),
        compiler_params=pltpu.CompilerParams(use_tc_tiling_on_sc=False),
        scratch_types=[
            pltpu.VMEM((rpt, CH), jnp.int32),
            pltpu.VMEM((rpt, CH), jnp.int32),
            pltpu.VMEM((K * CH, d), F32),
            pltpu.VMEM((K * CH, d), F32),
            pltpu.VMEM((K * CH, d), F32),
            pltpu.VMEM((K * CH, d), F32),
            pltpu.SemaphoreType.DMA,
            pltpu.SemaphoreType.DMA,
        ],
    )
    return k(zp, zs, e0_3d, e1_3d)


def _tc_layer1(deg_parts, x, w):
    n = x.shape[0]
    h_ = w.shape[1]

    def f(deg_ref, x_ref, w_ref, dis_ref, h_ref, g_ref):
        deg = deg_ref[0] + deg_ref[1] + 1.0
        dis = lax.rsqrt(deg)
        h = jnp.dot(x_ref[...], w_ref[...], preferred_element_type=F32)
        dis_ref[...] = dis
        h_ref[...] = h
        g_ref[...] = h * dis

    return pl.pallas_call(
        f,
        out_shape=[jax.ShapeDtypeStruct((n, 1), F32),
                   jax.ShapeDtypeStruct((n, h_), F32),
                   jax.ShapeDtypeStruct((n, h_), F32)],
    )(deg_parts, x, w)


def _tc_layer2(agg_parts, h1, dis, b1, w2):
    n, h_ = h1.shape

    def f(agg_ref, h_ref, dis_ref, b_ref, w_ref, h2_ref, g2_ref):
        dis_ = dis_ref[...]
        z = dis_ * (agg_ref[0] + agg_ref[1]) + (dis_ * dis_) * h_ref[...] \
            + b_ref[...]
        h2 = jnp.dot(z, w_ref[...], preferred_element_type=F32)
        h2_ref[...] = h2
        g2_ref[...] = h2 * dis_

    return pl.pallas_call(
        f,
        out_shape=[jax.ShapeDtypeStruct((n, h_), F32),
                   jax.ShapeDtypeStruct((n, h_), F32)],
    )(agg_parts, h1, dis, b1, w2)


def _tc_project(agg_parts, h2, dis, b2, wproj_half, wl1):
    n, h_ = h2.shape

    def f(agg_ref, h_ref, dis_ref, b_ref, wp_ref, wl1_ref, z_ref):
        dis_ = dis_ref[...]
        z2 = dis_ * (agg_ref[0] + agg_ref[1]) + (dis_ * dis_) * h_ref[...] \
            + b_ref[...]
        wc = jnp.dot(wp_ref[...], wl1_ref[...], preferred_element_type=F32)
        z_ref[...] = jnp.dot(z2, wc, preferred_element_type=F32)

    return pl.pallas_call(
        f,
        out_shape=jax.ShapeDtypeStruct((n, wl1.shape[1]), F32),
    )(agg_parts, h2, dis, b2, wproj_half, wl1)


def _tc_head(gp, gs, bproj, wl1, bl1, wl2, bl2):
    q, h_ = gp.shape
    bq = 10000
    grid = q // bq

    def f(gp_ref, gs_ref, bp_ref, wl1_ref, bl1_ref, wl2_ref, bl2_ref, o_ref):
        bc = jnp.dot(bp_ref[...], wl1_ref[...],
                     preferred_element_type=F32) + bl1_ref[...]
        hh = jnp.maximum(gp_ref[...] + gs_ref[...] + bc, 0.0)
        o_ref[...] = jnp.dot(hh, wl2_ref[...],
                             preferred_element_type=F32) + bl2_ref[...]

    return pl.pallas_call(
        f,
        grid=(grid,),
        in_specs=[
            pl.BlockSpec((bq, h_), lambda i: (i, 0)),
            pl.BlockSpec((bq, h_), lambda i: (i, 0)),
            pl.BlockSpec(bproj.shape, lambda i: (0, 0)),
            pl.BlockSpec(wl1.shape, lambda i: (0, 0)),
            pl.BlockSpec(bl1.shape, lambda i: (0, 0)),
            pl.BlockSpec(wl2.shape, lambda i: (0, 0)),
            pl.BlockSpec(bl2.shape, lambda i: (0, 0)),
        ],
        out_specs=pl.BlockSpec((bq, 1), lambda i: (i, 0)),
        out_shape=jax.ShapeDtypeStruct((q, 1), F32),
    )(gp, gs, bproj, wl1, bl1, wl2, bl2)


def _gcn_stack(x, src3d, dst3d, deg_parts, w1, b1, w2, b2, wproj_half, wl1,
               zeros_nd):
    n = x.shape[0]
    d = w1.shape[1]
    dis, h1, g1 = _tc_layer1(deg_parts, x, w1)
    agg1 = _agg_call(n, d, g1, src3d, dst3d, zeros_nd)
    h2, g2 = _tc_layer2(agg1, h1, dis, b1, w2)
    agg2 = _agg_call(n, d, g2, src3d, dst3d, zeros_nd)
    return _tc_project(agg2, h2, dis, b2, wproj_half, wl1)


def kernel(x_protein, x_substrate, edge_index_protein, edge_index_substrate,
           edges, Wp1, bp1, Wp2, bp2, Ws1, bs1, Ws2, bs2, Wproj, bproj,
           Wl1, bl1, Wl2, bl2):
    n = x_protein.shape[0]
    e = edge_index_protein.shape[1]
    q = edges.shape[1]
    d = Wp1.shape[1]
    i32 = jnp.int32

    rpt = e // (CH * NW)
    srcp = edge_index_protein[0].astype(i32).reshape(NW, rpt, CH)
    dstp = edge_index_protein[1].astype(i32).reshape(NW, rpt, CH)
    srcs = edge_index_substrate[0].astype(i32).reshape(NW, rpt, CH)
    dsts = edge_index_substrate[1].astype(i32).reshape(NW, rpt, CH)

    qrows = -(-q // CH)
    qrpt = -(-qrows // NW)
    qpad = NW * qrpt * CH
    e0 = jnp.concatenate(
        [edges[0].astype(i32), jnp.zeros((qpad - q,), i32)]).reshape(
            NW, qrpt, CH)
    e1 = jnp.concatenate(
        [edges[1].astype(i32), jnp.zeros((qpad - q,), i32)]).reshape(
            NW, qrpt, CH)

    zeros_nd = jnp.zeros((n, d), F32)

    dp0, dp1, ds0, ds1 = _deg_call(n, rpt, dstp, dsts)
    degp = jnp.stack([dp0, dp1]).reshape(NC, n, 1)
    degs = jnp.stack([ds0, ds1]).reshape(NC, n, 1)

    zp = _gcn_stack(x_protein, srcp, dstp, degp, Wp1, bp1.reshape(1, d),
                    Wp2, bp2.reshape(1, d), Wproj[:d], Wl1, zeros_nd)
    zs = _gcn_stack(x_substrate, srcs, dsts, degs, Ws1, bs1.reshape(1, d),
                    Ws2, bs2.reshape(1, d), Wproj[d:], Wl1, zeros_nd)

    gp_pad, gs_pad = _pair_gather_call(n, qpad, d, zp, zs, e0, e1)

    out = _tc_head(gp_pad[:q], gs_pad[:q], bproj.reshape(1, 2 * d), Wl1,
                   bl1.reshape(1, d), Wl2, bl2.reshape(1, 1))
    return out.reshape(q)


# speedup vs baseline: 1.0062x; 1.0062x over previous
import jax
import jax.numpy as jnp
from jax import lax
from jax.experimental import pallas as pl
from jax.experimental.pallas import tpu as pltpu
from jax.experimental.pallas import tpu_sc as plsc

F32 = jnp.float32
NC, NS = 2, 16
NW = NC * NS
CH = 80


def _sc_mesh():
    return plsc.VectorSubcoreMesh(
        core_axis_name="c", subcore_axis_name="s",
        num_cores=NC, num_subcores=NS)


def _deg_call(n, rpt, dstp3d, dsts3d):
    blk = (n // NS) // 8 * 8
    tail = n - NS * blk
    nred = blk // 16
    inner = CH // 16

    def body(dstp, dsts, op0, op1, os0, os1,
             histp, hists, ip_all, is_all, red, redt, obuf, obuft,
             stgp, stgs, sem):
        c = lax.axis_index("c")
        s = lax.axis_index("s")
        wid = s * NC + c
        z16 = jnp.zeros((16,), F32)

        def zloop(i, carry):
            histp[pl.ds(i * 16, 16)] = z16
            hists[pl.ds(i * 16, 16)] = z16
            return carry
        lax.fori_loop(0, n // 16, zloop, 0)

        pltpu.sync_copy(dstp.at[wid], ip_all)
        pltpu.sync_copy(dsts.at[wid], is_all)

        for idx_all, hist in ((ip_all, histp), (is_all, hists)):
            def hloop(r, carry):
                for j in range(inner):
                    x = idx_all[r, pl.ds(j * 16, 16)]
                    cnt, last = plsc.scan_count(x)
                    plsc.addupdate_scatter(hist, [x], cnt.astype(F32),
                                           mask=last)
                return carry
            lax.fori_loop(0, rpt, hloop, 0)

        st_s = pl.multiple_of(s * n, 8)
        pltpu.sync_copy(histp, stgp.at[pl.ds(st_s, n)])
        pltpu.sync_copy(hists, stgs.at[pl.ds(st_s, n)])
        plsc.subcore_barrier()

        st = pl.multiple_of(s * blk, 8)
        for stg, o0, o1 in ((stgp, op0, op1), (stgs, os0, os1)):
            descs = [
                pltpu.async_copy(
                    stg.at[pl.ds(pl.multiple_of(k * n + st, 8), blk)],
                    red.at[k], sem)
                for k in range(NS)
            ]
            for de in descs:
                de.wait()

            def rloop(j, carry):
                v = red[0, pl.ds(j * 16, 16)]
                for k in range(1, NS):
                    v = v + red[k, pl.ds(j * 16, 16)]
                obuf[pl.ds(j * 16, 16)] = v
                return carry
            lax.fori_loop(0, nred, rloop, 0)

            @pl.when(c == 0)
            def _():
                pltpu.sync_copy(obuf, o0.at[pl.ds(st, blk)])

            @pl.when(c == 1)
            def _():
                pltpu.sync_copy(obuf, o1.at[pl.ds(st, blk)])

            @pl.when(s == NS - 1)
            def _():
                if tail:
                    tdescs = [
                        pltpu.async_copy(
                            stg.at[pl.ds(k * n + NS * blk, tail)],
                            redt.at[k], sem)
                        for k in range(NS)
                    ]
                    for de in tdescs:
                        de.wait()
                    v = redt[0, :]
                    for k in range(1, NS):
                        v = v + redt[k, :]
                    obuft[...] = v

                    @pl.when(c == 0)
                    def _():
                        pltpu.sync_copy(obuft, o0.at[pl.ds(NS * blk, tail)])

                    @pl.when(c == 1)
                    def _():
                        pltpu.sync_copy(obuft, o1.at[pl.ds(NS * blk, tail)])

    k = pl.kernel(
        body,
        out_type=[jax.ShapeDtypeStruct((n,), F32)] * 4,
        mesh=_sc_mesh(),
        compiler_params=pltpu.CompilerParams(use_tc_tiling_on_sc=False,
                                             needs_layout_passes=False),
        scratch_types=[
            pltpu.VMEM((n,), F32),
            pltpu.VMEM((n,), F32),
            pltpu.VMEM((rpt, CH), jnp.int32),
            pltpu.VMEM((rpt, CH), jnp.int32),
            pltpu.VMEM((NS, blk), F32),
            pltpu.VMEM((NS, 16), F32),
            pltpu.VMEM((blk,), F32),
            pltpu.VMEM((16,), F32),
            pltpu.VMEM_SHARED((NS * n,), F32),
            pltpu.VMEM_SHARED((NS * n,), F32),
            pltpu.SemaphoreType.DMA,
        ],
    )
    return k(dstp3d, dsts3d)


def _agg_call(n, d, g_nd, src3d, dst3d, zeros_nd):
    rpt = src3d.shape[1]
    K = 5
    nstg = rpt // K
    blk = (n // NS) // 8 * 8
    tail = n - NS * blk

    def body(g_hbm, src_h, dst_h, zn, out, acc,
             si_all, di_all, rows_a, rows_b, semg, sems):
        c = lax.axis_index("c")
        s = lax.axis_index("s")
        wid = s * NC + c
        rows = (rows_a, rows_b)

        st = pl.multiple_of(s * blk, 8)
        pltpu.sync_copy(zn.at[pl.ds(st, blk)], acc.at[pl.ds(st, blk)])

        @pl.when(s == NS - 1)
        def _():
            if tail:
                pltpu.sync_copy(zn.at[pl.ds(NS * blk, tail)],
                                acc.at[pl.ds(NS * blk, tail)])
        pltpu.sync_copy(src_h.at[wid], si_all)
        pltpu.sync_copy(dst_h.at[wid], di_all)
        plsc.subcore_barrier()

        def fire_gathers(t):
            dd = t % 2
            return [
                pltpu.async_copy(g_hbm.at[si_all.at[t * K + j]],
                                 rows[dd].at[j], semg)
                for j in range(K)
            ]

        def fire_scatters(t):
            dd = t % 2
            return [
                pltpu.async_copy(rows[dd].at[j],
                                 acc.at[di_all.at[t * K + j]], sems, add=True)
                for j in range(K)
            ]

        gat = {0: fire_gathers(0)}
        sca = {}
        for t in range(nstg):
            for de in gat.pop(t):
                de.wait()
            if t >= 1:
                for de in sca.pop(t - 1):
                    de.wait()
            if t + 1 < nstg:
                gat[t + 1] = fire_gathers(t + 1)
            sca[t] = fire_scatters(t)
        for de in sca.pop(nstg - 1):
            de.wait()

        plsc.subcore_barrier()
        pltpu.sync_copy(acc.at[pl.ds(st, blk)], out.at[c, pl.ds(st, blk)])

        @pl.when(s == NS - 1)
        def _():
            if tail:
                pltpu.sync_copy(acc.at[pl.ds(NS * blk, tail)],
                                out.at[c, pl.ds(NS * blk, tail)])

    k = pl.kernel(
        body,
        out_type=[jax.ShapeDtypeStruct((NC, n, d), F32)],
        mesh=_sc_mesh(),
        compiler_params=pltpu.CompilerParams(use_tc_tiling_on_sc=False),
        scratch_types=[
            pltpu.VMEM_SHARED((n, d), F32),
            pltpu.VMEM((rpt, CH), jnp.int32),
            pltpu.VMEM((rpt, CH), jnp.int32),
            pltpu.VMEM((K, CH, d), F32),
            pltpu.VMEM((K, CH, d), F32),
            pltpu.SemaphoreType.DMA,
            pltpu.SemaphoreType.DMA,
        ],
    )
    (out,) = k(g_nd, src3d, dst3d, zeros_nd)
    return out


def _pair_gather_call(n, qpad, d, zp, zs, e0_3d, e1_3d):
    rpt = e0_3d.shape[1]
    K = 2
    nstg = rpt // K

    def body(zp_h, zs_h, e0_h, e1_h, gp, gs,
             i0_all, i1_all, r0a, r0b, r0c, r1a, r1b, r1c, semg, semw):
        c = lax.axis_index("c")
        s = lax.axis_index("s")
        wid = s * NC + c
        r0 = (r0a, r0b, r0c)
        r1 = (r1a, r1b, r1c)

        pltpu.sync_copy(e0_h.at[wid], i0_all)
        pltpu.sync_copy(e1_h.at[wid], i1_all)

        def fire_gathers(t):
            dd = t % 3
            descs = []
            for j in range(K):
                descs.append(pltpu.async_copy(
                    zp_h.at[i0_all.at[t * K + j]],
                    r0[dd].at[pl.ds(j * CH, CH)], semg))
                descs.append(pltpu.async_copy(
                    zs_h.at[i1_all.at[t * K + j]],
                    r1[dd].at[pl.ds(j * CH, CH)], semg))
            return descs

        def fire_writes(t):
            dd = t % 3
            off = pl.multiple_of((wid * rpt + t * K) * CH, 8)
            return [
                pltpu.async_copy(r0[dd], gp.at[pl.ds(off, K * CH)], semw),
                pltpu.async_copy(r1[dd], gs.at[pl.ds(off, K * CH)], semw),
            ]

        gat = {0: fire_gathers(0), 1: fire_gathers(1)}
        wrt = {}
        for t in range(nstg):
            for de in gat.pop(t):
                de.wait()
            if t >= 1:
                for de in wrt.pop(t - 1):
                    de.wait()
            if t + 2 < nstg:
                gat[t + 2] = fire_gathers(t + 2)
            wrt[t] = fire_writes(t)
        for de in wrt.pop(nstg - 1):
            de.wait()

    k = pl.kernel(
        body,
        out_type=[jax.ShapeDtypeStruct((qpad, d), F32),
                  jax.ShapeDtypeStruct((qpad, d), F32)],
        mesh=_sc_mesh(),
        compiler_params=pltpu.CompilerParams(use_tc_tiling_on_sc=False),
        scratch_types=[
            pltpu.VMEM((rpt, CH), jnp.int32),
            pltpu.VMEM((rpt, CH), jnp.int32),
            pltpu.VMEM((K * CH, d), F32),
            pltpu.VMEM((K * CH, d), F32),
            pltpu.VMEM((K * CH, d), F32),
            pltpu.VMEM((K * CH, d), F32),
            pltpu.VMEM((K * CH, d), F32),
            pltpu.VMEM((K * CH, d), F32),
            pltpu.SemaphoreType.DMA,
            pltpu.SemaphoreType.DMA,
        ],
    )
    return k(zp, zs, e0_3d, e1_3d)


def _tc_layer1(deg_parts, x, w):
    n = x.shape[0]
    h_ = w.shape[1]

    def f(deg_ref, x_ref, w_ref, dis_ref, h_ref, g_ref):
        deg = deg_ref[0] + deg_ref[1] + 1.0
        dis = lax.rsqrt(deg)
        h = jnp.dot(x_ref[...], w_ref[...], preferred_element_type=F32)
        dis_ref[...] = dis
        h_ref[...] = h
        g_ref[...] = h * dis

    return pl.pallas_call(
        f,
        out_shape=[jax.ShapeDtypeStruct((n, 1), F32),
                   jax.ShapeDtypeStruct((n, h_), F32),
                   jax.ShapeDtypeStruct((n, h_), F32)],
    )(deg_parts, x, w)


def _tc_layer2(agg_parts, h1, dis, b1, w2):
    n, h_ = h1.shape

    def f(agg_ref, h_ref, dis_ref, b_ref, w_ref, h2_ref, g2_ref):
        dis_ = dis_ref[...]
        z = dis_ * (agg_ref[0] + agg_ref[1]) + (dis_ * dis_) * h_ref[...] \
            + b_ref[...]
        h2 = jnp.dot(z, w_ref[...], preferred_element_type=F32)
        h2_ref[...] = h2
        g2_ref[...] = h2 * dis_

    return pl.pallas_call(
        f,
        out_shape=[jax.ShapeDtypeStruct((n, h_), F32),
                   jax.ShapeDtypeStruct((n, h_), F32)],
    )(agg_parts, h1, dis, b1, w2)


def _tc_project(agg_parts, h2, dis, b2, wproj_half, wl1):
    n, h_ = h2.shape

    def f(agg_ref, h_ref, dis_ref, b_ref, wp_ref, wl1_ref, z_ref):
        dis_ = dis_ref[...]
        z2 = dis_ * (agg_ref[0] + agg_ref[1]) + (dis_ * dis_) * h_ref[...] \
            + b_ref[...]
        wc = jnp.dot(wp_ref[...], wl1_ref[...], preferred_element_type=F32)
        z_ref[...] = jnp.dot(z2, wc, preferred_element_type=F32)

    return pl.pallas_call(
        f,
        out_shape=jax.ShapeDtypeStruct((n, wl1.shape[1]), F32),
    )(agg_parts, h2, dis, b2, wproj_half, wl1)


def _tc_head(gp, gs, bproj, wl1, bl1, wl2, bl2):
    q, h_ = gp.shape
    bq = 10000
    grid = q // bq

    def f(gp_ref, gs_ref, bp_ref, wl1_ref, bl1_ref, wl2_ref, bl2_ref, o_ref):
        bc = jnp.dot(bp_ref[...], wl1_ref[...],
                     preferred_element_type=F32) + bl1_ref[...]
        hh = jnp.maximum(gp_ref[...] + gs_ref[...] + bc, 0.0)
        o_ref[...] = jnp.dot(hh, wl2_ref[...],
                             preferred_element_type=F32) + bl2_ref[...]

    return pl.pallas_call(
        f,
        grid=(grid,),
        in_specs=[
            pl.BlockSpec((bq, h_), lambda i: (i, 0)),
            pl.BlockSpec((bq, h_), lambda i: (i, 0)),
            pl.BlockSpec(bproj.shape, lambda i: (0, 0)),
            pl.BlockSpec(wl1.shape, lambda i: (0, 0)),
            pl.BlockSpec(bl1.shape, lambda i: (0, 0)),
            pl.BlockSpec(wl2.shape, lambda i: (0, 0)),
            pl.BlockSpec(bl2.shape, lambda i: (0, 0)),
        ],
        out_specs=pl.BlockSpec((bq, 1), lambda i: (i, 0)),
        out_shape=jax.ShapeDtypeStruct((q, 1), F32),
    )(gp, gs, bproj, wl1, bl1, wl2, bl2)


def _gcn_stack(x, src3d, dst3d, deg_parts, w1, b1, w2, b2, wproj_half, wl1,
               zeros_nd):
    n = x.shape[0]
    d = w1.shape[1]
    dis, h1, g1 = _tc_layer1(deg_parts, x, w1)
    agg1 = _agg_call(n, d, g1, src3d, dst3d, zeros_nd)
    h2, g2 = _tc_layer2(agg1, h1, dis, b1, w2)
    agg2 = _agg_call(n, d, g2, src3d, dst3d, zeros_nd)
    return _tc_project(agg2, h2, dis, b2, wproj_half, wl1)


def kernel(x_protein, x_substrate, edge_index_protein, edge_index_substrate,
           edges, Wp1, bp1, Wp2, bp2, Ws1, bs1, Ws2, bs2, Wproj, bproj,
           Wl1, bl1, Wl2, bl2):
    n = x_protein.shape[0]
    e = edge_index_protein.shape[1]
    q = edges.shape[1]
    d = Wp1.shape[1]
    i32 = jnp.int32

    rpt = e // (CH * NW)
    srcp = edge_index_protein[0].astype(i32).reshape(NW, rpt, CH)
    dstp = edge_index_protein[1].astype(i32).reshape(NW, rpt, CH)
    srcs = edge_index_substrate[0].astype(i32).reshape(NW, rpt, CH)
    dsts = edge_index_substrate[1].astype(i32).reshape(NW, rpt, CH)

    qrows = -(-q // CH)
    qrpt = -(-qrows // NW)
    qpad = NW * qrpt * CH
    e0 = jnp.concatenate(
        [edges[0].astype(i32), jnp.zeros((qpad - q,), i32)]).reshape(
            NW, qrpt, CH)
    e1 = jnp.concatenate(
        [edges[1].astype(i32), jnp.zeros((qpad - q,), i32)]).reshape(
            NW, qrpt, CH)

    zeros_nd = jnp.zeros((n, d), F32)

    dp0, dp1, ds0, ds1 = _deg_call(n, rpt, dstp, dsts)
    degp = jnp.stack([dp0, dp1]).reshape(NC, n, 1)
    degs = jnp.stack([ds0, ds1]).reshape(NC, n, 1)

    zp = _gcn_stack(x_protein, srcp, dstp, degp, Wp1, bp1.reshape(1, d),
                    Wp2, bp2.reshape(1, d), Wproj[:d], Wl1, zeros_nd)
    zs = _gcn_stack(x_substrate, srcs, dsts, degs, Ws1, bs1.reshape(1, d),
                    Ws2, bs2.reshape(1, d), Wproj[d:], Wl1, zeros_nd)

    gp_pad, gs_pad = _pair_gather_call(n, qpad, d, zp, zs, e0, e1)

    out = _tc_head(gp_pad[:q], gs_pad[:q], bproj.reshape(1, 2 * d), Wl1,
                   bl1.reshape(1, d), Wl2, bl2.reshape(1, 1))
    return out.reshape(q)


# speedup vs baseline: 1.0619x; 1.0553x over previous
import jax
import jax.numpy as jnp
from jax import lax
from jax.experimental import pallas as pl
from jax.experimental.pallas import tpu as pltpu
from jax.experimental.pallas import tpu_sc as plsc

F32 = jnp.float32
NC, NS = 2, 16
NW = NC * NS
CH = 80


def _sc_mesh():
    return plsc.VectorSubcoreMesh(
        core_axis_name="c", subcore_axis_name="s",
        num_cores=NC, num_subcores=NS)


def _deg_call(n, rpt, dstp3d, dsts3d):
    blk = (n // NS) // 8 * 8
    tail = n - NS * blk
    nred = blk // 16
    inner = CH // 16

    def body(dstp, dsts, op0, op1, os0, os1,
             histp, hists, ip_all, is_all, red, redt, obuf, obuft,
             stgp, stgs, sem):
        c = lax.axis_index("c")
        s = lax.axis_index("s")
        wid = s * NC + c
        z16 = jnp.zeros((16,), F32)

        def zloop(i, carry):
            histp[pl.ds(i * 16, 16)] = z16
            hists[pl.ds(i * 16, 16)] = z16
            return carry
        lax.fori_loop(0, n // 16, zloop, 0)

        pltpu.sync_copy(dstp.at[wid], ip_all)
        pltpu.sync_copy(dsts.at[wid], is_all)

        for idx_all, hist in ((ip_all, histp), (is_all, hists)):
            def hloop(r, carry):
                for j in range(inner):
                    x = idx_all[r, pl.ds(j * 16, 16)]
                    cnt, last = plsc.scan_count(x)
                    plsc.addupdate_scatter(hist, [x], cnt.astype(F32),
                                           mask=last)
                return carry
            lax.fori_loop(0, rpt, hloop, 0)

        st_s = pl.multiple_of(s * n, 8)
        pltpu.sync_copy(histp, stgp.at[pl.ds(st_s, n)])
        pltpu.sync_copy(hists, stgs.at[pl.ds(st_s, n)])
        plsc.subcore_barrier()

        st = pl.multiple_of(s * blk, 8)
        for stg, o0, o1 in ((stgp, op0, op1), (stgs, os0, os1)):
            descs = [
                pltpu.async_copy(
                    stg.at[pl.ds(pl.multiple_of(k * n + st, 8), blk)],
                    red.at[k], sem)
                for k in range(NS)
            ]
            for de in descs:
                de.wait()

            def rloop(j, carry):
                v = red[0, pl.ds(j * 16, 16)]
                for k in range(1, NS):
                    v = v + red[k, pl.ds(j * 16, 16)]
                obuf[pl.ds(j * 16, 16)] = v
                return carry
            lax.fori_loop(0, nred, rloop, 0)

            @pl.when(c == 0)
            def _():
                pltpu.sync_copy(obuf, o0.at[pl.ds(st, blk)])

            @pl.when(c == 1)
            def _():
                pltpu.sync_copy(obuf, o1.at[pl.ds(st, blk)])

            @pl.when(s == NS - 1)
            def _():
                if tail:
                    tdescs = [
                        pltpu.async_copy(
                            stg.at[pl.ds(k * n + NS * blk, tail)],
                            redt.at[k], sem)
                        for k in range(NS)
                    ]
                    for de in tdescs:
                        de.wait()
                    v = redt[0, :]
                    for k in range(1, NS):
                        v = v + redt[k, :]
                    obuft[...] = v

                    @pl.when(c == 0)
                    def _():
                        pltpu.sync_copy(obuft, o0.at[pl.ds(NS * blk, tail)])

                    @pl.when(c == 1)
                    def _():
                        pltpu.sync_copy(obuft, o1.at[pl.ds(NS * blk, tail)])

    k = pl.kernel(
        body,
        out_type=[jax.ShapeDtypeStruct((n,), F32)] * 4,
        mesh=_sc_mesh(),
        compiler_params=pltpu.CompilerParams(use_tc_tiling_on_sc=False,
                                             needs_layout_passes=False),
        scratch_types=[
            pltpu.VMEM((n,), F32),
            pltpu.VMEM((n,), F32),
            pltpu.VMEM((rpt, CH), jnp.int32),
            pltpu.VMEM((rpt, CH), jnp.int32),
            pltpu.VMEM((NS, blk), F32),
            pltpu.VMEM((NS, 16), F32),
            pltpu.VMEM((blk,), F32),
            pltpu.VMEM((16,), F32),
            pltpu.VMEM_SHARED((NS * n,), F32),
            pltpu.VMEM_SHARED((NS * n,), F32),
            pltpu.SemaphoreType.DMA,
        ],
    )
    return k(dstp3d, dsts3d)


def _agg_call(n, d, g_nd, src3d, dst3d, zeros_nd):
    rpt = src3d.shape[1]
    K = 5
    nstg = rpt // K
    blk = (n // NS) // 8 * 8
    tail = n - NS * blk

    def body(g_hbm, src_h, dst_h, zn, out, acc,
             si_all, di_all, rows_a, rows_b, semg, sems):
        c = lax.axis_index("c")
        s = lax.axis_index("s")
        wid = s * NC + c
        rows = (rows_a, rows_b)

        st = pl.multiple_of(s * blk, 8)
        pltpu.sync_copy(zn.at[pl.ds(st, blk)], acc.at[pl.ds(st, blk)])

        @pl.when(s == NS - 1)
        def _():
            if tail:
                pltpu.sync_copy(zn.at[pl.ds(NS * blk, tail)],
                                acc.at[pl.ds(NS * blk, tail)])
        pltpu.sync_copy(src_h.at[wid], si_all)
        pltpu.sync_copy(dst_h.at[wid], di_all)
        plsc.subcore_barrier()

        def fire_gathers(t):
            dd = t % 2
            return [
                pltpu.async_copy(g_hbm.at[si_all.at[t * K + j]],
                                 rows[dd].at[j], semg)
                for j in range(K)
            ]

        def fire_scatters(t):
            dd = t % 2
            return [
                pltpu.async_copy(rows[dd].at[j],
                                 acc.at[di_all.at[t * K + j]], sems, add=True)
                for j in range(K)
            ]

        gat = {0: fire_gathers(0)}
        sca = {}
        for t in range(nstg):
            for de in gat.pop(t):
                de.wait()
            if t >= 1:
                for de in sca.pop(t - 1):
                    de.wait()
            if t + 1 < nstg:
                gat[t + 1] = fire_gathers(t + 1)
            sca[t] = fire_scatters(t)
        for de in sca.pop(nstg - 1):
            de.wait()

        plsc.subcore_barrier()
        pltpu.sync_copy(acc.at[pl.ds(st, blk)], out.at[c, pl.ds(st, blk)])

        @pl.when(s == NS - 1)
        def _():
            if tail:
                pltpu.sync_copy(acc.at[pl.ds(NS * blk, tail)],
                                out.at[c, pl.ds(NS * blk, tail)])

    k = pl.kernel(
        body,
        out_type=[jax.ShapeDtypeStruct((NC, n, d), F32)],
        mesh=_sc_mesh(),
        compiler_params=pltpu.CompilerParams(use_tc_tiling_on_sc=False),
        scratch_types=[
            pltpu.VMEM_SHARED((n, d), F32),
            pltpu.VMEM((rpt, CH), jnp.int32),
            pltpu.VMEM((rpt, CH), jnp.int32),
            pltpu.VMEM((K, CH, d), F32),
            pltpu.VMEM((K, CH, d), F32),
            pltpu.SemaphoreType.DMA,
            pltpu.SemaphoreType.DMA,
        ],
    )
    (out,) = k(g_nd, src3d, dst3d, zeros_nd)
    return out


def _pair_gather_call(n, qpad, d, zp, zs, e0_3d, e1_3d):
    rpt = e0_3d.shape[1]
    K = 2
    nstg = rpt // K

    def body(zp_h, zs_h, e0_h, e1_h, gp, gs,
             i0_all, i1_all, r0a, r0b, r0c, r1a, r1b, r1c, semg, semw):
        c = lax.axis_index("c")
        s = lax.axis_index("s")
        wid = s * NC + c
        r0 = (r0a, r0b, r0c)
        r1 = (r1a, r1b, r1c)

        pltpu.sync_copy(e0_h.at[wid], i0_all)
        pltpu.sync_copy(e1_h.at[wid], i1_all)

        def fire_gathers(t):
            dd = t % 3
            descs = []
            for j in range(K):
                descs.append(pltpu.async_copy(
                    zp_h.at[i0_all.at[t * K + j]],
                    r0[dd].at[pl.ds(j * CH, CH)], semg))
                descs.append(pltpu.async_copy(
                    zs_h.at[i1_all.at[t * K + j]],
                    r1[dd].at[pl.ds(j * CH, CH)], semg))
            return descs

        def fire_writes(t):
            dd = t % 3
            off = pl.multiple_of((wid * rpt + t * K) * CH, 8)
            return [
                pltpu.async_copy(r0[dd], gp.at[pl.ds(off, K * CH)], semw),
                pltpu.async_copy(r1[dd], gs.at[pl.ds(off, K * CH)], semw),
            ]

        gat = {0: fire_gathers(0), 1: fire_gathers(1)}
        wrt = {}
        for t in range(nstg):
            for de in gat.pop(t):
                de.wait()
            if t >= 1:
                for de in wrt.pop(t - 1):
                    de.wait()
            if t + 2 < nstg:
                gat[t + 2] = fire_gathers(t + 2)
            wrt[t] = fire_writes(t)
        for de in wrt.pop(nstg - 1):
            de.wait()

    k = pl.kernel(
        body,
        out_type=[jax.ShapeDtypeStruct((qpad, d), jnp.bfloat16),
                  jax.ShapeDtypeStruct((qpad, d), jnp.bfloat16)],
        mesh=_sc_mesh(),
        compiler_params=pltpu.CompilerParams(use_tc_tiling_on_sc=False),
        scratch_types=[
            pltpu.VMEM((rpt, CH), jnp.int32),
            pltpu.VMEM((rpt, CH), jnp.int32),
            pltpu.VMEM((K * CH, d), jnp.bfloat16),
            pltpu.VMEM((K * CH, d), jnp.bfloat16),
            pltpu.VMEM((K * CH, d), jnp.bfloat16),
            pltpu.VMEM((K * CH, d), jnp.bfloat16),
            pltpu.VMEM((K * CH, d), jnp.bfloat16),
            pltpu.VMEM((K * CH, d), jnp.bfloat16),
            pltpu.SemaphoreType.DMA,
            pltpu.SemaphoreType.DMA,
        ],
    )
    return k(zp, zs, e0_3d, e1_3d)


def _tc_layer1(deg_parts, x, w):
    n = x.shape[0]
    h_ = w.shape[1]

    def f(deg_ref, x_ref, w_ref, dis_ref, h_ref, g_ref):
        deg = deg_ref[0] + deg_ref[1] + 1.0
        dis = lax.rsqrt(deg)
        h = jnp.dot(x_ref[...], w_ref[...], preferred_element_type=F32)
        dis_ref[...] = dis
        h_ref[...] = h
        g_ref[...] = h * dis

    return pl.pallas_call(
        f,
        out_shape=[jax.ShapeDtypeStruct((n, 1), F32),
                   jax.ShapeDtypeStruct((n, h_), F32),
                   jax.ShapeDtypeStruct((n, h_), F32)],
    )(deg_parts, x, w)


def _tc_layer2(agg_parts, h1, dis, b1, w2):
    n, h_ = h1.shape

    def f(agg_ref, h_ref, dis_ref, b_ref, w_ref, h2_ref, g2_ref):
        dis_ = dis_ref[...]
        z = dis_ * (agg_ref[0] + agg_ref[1]) + (dis_ * dis_) * h_ref[...] \
            + b_ref[...]
        h2 = jnp.dot(z, w_ref[...], preferred_element_type=F32)
        h2_ref[...] = h2
        g2_ref[...] = h2 * dis_

    return pl.pallas_call(
        f,
        out_shape=[jax.ShapeDtypeStruct((n, h_), F32),
                   jax.ShapeDtypeStruct((n, h_), F32)],
    )(agg_parts, h1, dis, b1, w2)


def _tc_project(agg_parts, h2, dis, b2, wproj_half, wl1):
    n, h_ = h2.shape

    def f(agg_ref, h_ref, dis_ref, b_ref, wp_ref, wl1_ref, z_ref):
        dis_ = dis_ref[...]
        z2 = dis_ * (agg_ref[0] + agg_ref[1]) + (dis_ * dis_) * h_ref[...] \
            + b_ref[...]
        wc = jnp.dot(wp_ref[...], wl1_ref[...], preferred_element_type=F32)
        z_ref[...] = jnp.dot(z2, wc,
                             preferred_element_type=F32).astype(jnp.bfloat16)

    return pl.pallas_call(
        f,
        out_shape=jax.ShapeDtypeStruct((n, wl1.shape[1]), jnp.bfloat16),
    )(agg_parts, h2, dis, b2, wproj_half, wl1)


def _tc_head(gp, gs, bproj, wl1, bl1, wl2, bl2):
    q, h_ = gp.shape
    bq = 10000
    grid = q // bq

    def f(gp_ref, gs_ref, bp_ref, wl1_ref, bl1_ref, wl2_ref, bl2_ref, o_ref):
        bc = jnp.dot(bp_ref[...], wl1_ref[...],
                     preferred_element_type=F32) + bl1_ref[...]
        hh = jnp.maximum(gp_ref[...].astype(F32) + gs_ref[...].astype(F32) + bc, 0.0)
        o_ref[...] = jnp.dot(hh, wl2_ref[...],
                             preferred_element_type=F32) + bl2_ref[...]

    return pl.pallas_call(
        f,
        grid=(grid,),
        in_specs=[
            pl.BlockSpec((bq, h_), lambda i: (i, 0)),
            pl.BlockSpec((bq, h_), lambda i: (i, 0)),
            pl.BlockSpec(bproj.shape, lambda i: (0, 0)),
            pl.BlockSpec(wl1.shape, lambda i: (0, 0)),
            pl.BlockSpec(bl1.shape, lambda i: (0, 0)),
            pl.BlockSpec(wl2.shape, lambda i: (0, 0)),
            pl.BlockSpec(bl2.shape, lambda i: (0, 0)),
        ],
        out_specs=pl.BlockSpec((bq, 1), lambda i: (i, 0)),
        out_shape=jax.ShapeDtypeStruct((q, 1), F32),
    )(gp, gs, bproj, wl1, bl1, wl2, bl2)


def _gcn_stack(x, src3d, dst3d, deg_parts, w1, b1, w2, b2, wproj_half, wl1,
               zeros_nd):
    n = x.shape[0]
    d = w1.shape[1]
    dis, h1, g1 = _tc_layer1(deg_parts, x, w1)
    agg1 = _agg_call(n, d, g1, src3d, dst3d, zeros_nd)
    h2, g2 = _tc_layer2(agg1, h1, dis, b1, w2)
    agg2 = _agg_call(n, d, g2, src3d, dst3d, zeros_nd)
    return _tc_project(agg2, h2, dis, b2, wproj_half, wl1)


def kernel(x_protein, x_substrate, edge_index_protein, edge_index_substrate,
           edges, Wp1, bp1, Wp2, bp2, Ws1, bs1, Ws2, bs2, Wproj, bproj,
           Wl1, bl1, Wl2, bl2):
    n = x_protein.shape[0]
    e = edge_index_protein.shape[1]
    q = edges.shape[1]
    d = Wp1.shape[1]
    i32 = jnp.int32

    rpt = e // (CH * NW)
    srcp = edge_index_protein[0].astype(i32).reshape(NW, rpt, CH)
    dstp = edge_index_protein[1].astype(i32).reshape(NW, rpt, CH)
    srcs = edge_index_substrate[0].astype(i32).reshape(NW, rpt, CH)
    dsts = edge_index_substrate[1].astype(i32).reshape(NW, rpt, CH)

    qrows = -(-q // CH)
    qrpt = -(-qrows // NW)
    qpad = NW * qrpt * CH
    e0 = jnp.concatenate(
        [edges[0].astype(i32), jnp.zeros((qpad - q,), i32)]).reshape(
            NW, qrpt, CH)
    e1 = jnp.concatenate(
        [edges[1].astype(i32), jnp.zeros((qpad - q,), i32)]).reshape(
            NW, qrpt, CH)

    zeros_nd = jnp.zeros((n, d), F32)

    dp0, dp1, ds0, ds1 = _deg_call(n, rpt, dstp, dsts)
    degp = jnp.stack([dp0, dp1]).reshape(NC, n, 1)
    degs = jnp.stack([ds0, ds1]).reshape(NC, n, 1)

    zp = _gcn_stack(x_protein, srcp, dstp, degp, Wp1, bp1.reshape(1, d),
                    Wp2, bp2.reshape(1, d), Wproj[:d], Wl1, zeros_nd)
    zs = _gcn_stack(x_substrate, srcs, dsts, degs, Ws1, bs1.reshape(1, d),
                    Ws2, bs2.reshape(1, d), Wproj[d:], Wl1, zeros_nd)

    gp_pad, gs_pad = _pair_gather_call(n, qpad, d, zp, zs, e0, e1)

    out = _tc_head(gp_pad[:q], gs_pad[:q], bproj.reshape(1, 2 * d), Wl1,
                   bl1.reshape(1, d), Wl2, bl2.reshape(1, 1))
    return out.reshape(q)
